# Initial kernel scaffold; baseline (speedup 1.0000x reference)
import jax
import jax.numpy as jnp
from jax.experimental import pallas as pl


def kernel(x, table):
    raise NotImplementedError("write your pallas kernel here")


# speedup vs baseline: 1.2950x; 1.2950x over previous
import functools
import math

import jax
import jax.numpy as jnp
from jax import lax
from jax.experimental import pallas as pl
from jax.experimental.pallas import tpu as pltpu
from jax.experimental.pallas import tpu_sc as plsc

D_EMBED = 32
VOCAB = 1000000
BATCH = 16384
HIST = 20
SCALE = math.sqrt(D_EMBED)

NC, NS, L = 2, 16, 16
NW = NC * NS
B = BATCH * HIST
IDX_ROWS = B // 128
ROWS_PER_W = IDX_ROWS // NW
CHUNK_ROWS = 8
CHUNK = CHUNK_ROWS * 128
NCHUNK = ROWS_PER_W // CHUNK_ROWS


def _emb_body(x_hbm, table_hbm, out_hbm, idx_v, rows_v, sem):
    wid = lax.axis_index("s") * NC + lax.axis_index("c")
    row_base = wid * ROWS_PER_W
    out_base = wid * ROWS_PER_W * 128

    pltpu.sync_copy(x_hbm.at[pl.ds(row_base, ROWS_PER_W)], idx_v)

    @pl.loop(0, NCHUNK)
    def _chunk(g):
        cps = []
        for j in range(CHUNK_ROWS):
            cps.append(pltpu.async_copy(
                table_hbm.at[idx_v.at[g * CHUNK_ROWS + j]],
                rows_v.at[pl.ds(j * 128, 128)],
                sem,
            ))
        for cp in cps:
            cp.wait()

        @pl.loop(0, CHUNK, unroll=4)
        def _scale(i):
            rows_v[i, pl.ds(0, L)] = rows_v[i, pl.ds(0, L)] * SCALE
            rows_v[i, pl.ds(L, L)] = rows_v[i, pl.ds(L, L)] * SCALE

        pltpu.sync_copy(rows_v, out_hbm.at[pl.ds(out_base + g * CHUNK, CHUNK)])


@functools.partial(jax.jit, static_argnames=())
def _emb(x2d, table):
    mesh = plsc.VectorSubcoreMesh(core_axis_name="c", subcore_axis_name="s")
    f = functools.partial(
        pl.kernel,
        out_type=jax.ShapeDtypeStruct((B, D_EMBED), jnp.float32),
        mesh=mesh,
        scratch_types=[
            pltpu.VMEM((ROWS_PER_W, 128), jnp.int32),
            pltpu.VMEM((CHUNK, D_EMBED), jnp.float32),
            pltpu.SemaphoreType.DMA,
        ],
        compiler_params=pltpu.CompilerParams(use_tc_tiling_on_sc=False),
    )(_emb_body)
    return f(x2d, table)


def kernel(x, table):
    x2d = x.reshape(IDX_ROWS, 128)
    out = _emb(x2d, table)
    return out.reshape(BATCH, HIST, D_EMBED)
